# baseline (speedup 1.0000x reference)
import functools

import jax
import jax.numpy as jnp
import numpy as np
from jax.experimental import pallas as pl

TASKS, N, IN_DIM, OUT_DIM, K = 4, 10000, 32, 32, 3
TC = TASKS * IN_DIM
BM = 400


def _pass1_body(L_ref, xf_ref, y1_ref):
    y1_ref[...] = jnp.dot(L_ref[...], xf_ref[...],
                          preferred_element_type=jnp.float32)


def _pass2_body(L_ref, y1_ref, xf_ref, w_ref, b_ref, out_ref):
    z = jnp.dot(L_ref[...], y1_ref[...], preferred_element_type=jnp.float32)
    i = pl.program_id(0)
    xf_t = xf_ref[...]
    y1_t = y1_ref[pl.ds(i * BM, BM), :]
    w0 = w_ref[0]
    w1 = w_ref[1]
    w2 = w_ref[2]
    acc = jnp.dot(xf_t, w0 - w2, preferred_element_type=jnp.float32)
    acc += jnp.dot(y1_t, w1, preferred_element_type=jnp.float32)
    acc += jnp.dot(2.0 * z, w2, preferred_element_type=jnp.float32)
    out_ref[...] = acc + b_ref[...]


@jax.jit
def kernel(x, L_cheb, weight, bias):
    tasks, n, c = x.shape
    kdeg = weight.shape[1]
    tc = tasks * c
    xf = jnp.transpose(x, (1, 0, 2)).reshape(n, tc)
    eye = jnp.eye(tasks, dtype=weight.dtype)
    wbd = jnp.einsum('ts,tkio->ksito', eye, weight).reshape(
        kdeg, tasks * c, tasks * weight.shape[-1])
    bias_flat = bias.reshape(1, tasks * bias.shape[-1])

    grid = (n // BM,)

    y1 = pl.pallas_call(
        _pass1_body,
        grid=grid,
        in_specs=[
            pl.BlockSpec((BM, n), lambda i: (i, 0)),
            pl.BlockSpec((n, tc), lambda i: (0, 0)),
        ],
        out_specs=pl.BlockSpec((BM, tc), lambda i: (i, 0)),
        out_shape=jax.ShapeDtypeStruct((n, tc), jnp.float32),
    )(L_cheb, xf)

    out_f = pl.pallas_call(
        _pass2_body,
        grid=grid,
        in_specs=[
            pl.BlockSpec((BM, n), lambda i: (i, 0)),
            pl.BlockSpec((n, tc), lambda i: (0, 0)),
            pl.BlockSpec((BM, tc), lambda i: (i, 0)),
            pl.BlockSpec(wbd.shape, lambda i: (0, 0, 0)),
            pl.BlockSpec((1, tc), lambda i: (0, 0)),
        ],
        out_specs=pl.BlockSpec((BM, tc), lambda i: (i, 0)),
        out_shape=jax.ShapeDtypeStruct((n, tc), jnp.float32),
    )(L_cheb, y1, xf, wbd, bias_flat)

    return jnp.transpose(out_f.reshape(n, tasks, c), (1, 0, 2))
